# baseline (speedup 1.0000x reference)
import jax
import jax.numpy as jnp
from jax import lax
from jax.experimental import pallas as pl
from jax.experimental.pallas import tpu as pltpu
from jax.experimental.pallas import tpu_sc as plsc

N_NODES = 10000
N_EDGES = 160000
D = 256
NC = 2
NS = 16
EPW = 5120
EPAD = EPW * NS * NC
CH = 128
NCH = EPW // CH
NCH0 = 20
NCH1 = 2 * NCH - NCH0
NCHMAX = NCH1
RPW = 624
RTAIL = N_NODES - RPW * NS
AROWS = N_NODES + 8
DRPW = 632
DEGROWS = DRPW * NS
ROWBLK = 2000
GRID = N_NODES // ROWBLK


def _sc_degree_body(dst_ref, ones_ref, zeros_ref, deg_ref, ones_v, didx_v, hist_sh):
    c = lax.axis_index("c")
    s = lax.axis_index("s")
    pltpu.sync_copy(zeros_ref.at[pl.ds(s * DRPW, DRPW)], hist_sh.at[pl.ds(s * DRPW, DRPW)])
    pltpu.sync_copy(ones_ref, ones_v)
    plsc.subcore_barrier()

    def chunk(k, carry):
        base = s * (EPW * NC) + k * CH
        pltpu.sync_copy(dst_ref.at[pl.ds(base, CH)], didx_v)
        pltpu.sync_copy(ones_v, hist_sh.at[didx_v], add=True)
        return carry

    lax.fori_loop(0, NCH * NC, chunk, 0)
    plsc.subcore_barrier()

    @pl.when(c == 0)
    def _():
        pltpu.sync_copy(hist_sh.at[pl.ds(s * RPW, RPW)], deg_ref.at[pl.ds(s * RPW, RPW)])

    @pl.when((c == 0) & (s == NS - 1))
    def _():
        pltpu.sync_copy(hist_sh.at[pl.ds(RPW * NS, RTAIL)], deg_ref.at[pl.ds(RPW * NS, RTAIL)])


def _sc_degree(dstp, ones16, zeros16):
    f = pl.kernel(
        _sc_degree_body,
        out_type=jax.ShapeDtypeStruct((N_NODES, 16), jnp.float32),
        mesh=plsc.VectorSubcoreMesh(core_axis_name="c", subcore_axis_name="s"),
        scratch_types=[
            pltpu.VMEM((CH, 16), jnp.float32),
            pltpu.VMEM((CH,), jnp.int32),
            pltpu.VMEM_SHARED((DEGROWS, 16), jnp.float32),
        ],
        compiler_params=pltpu.CompilerParams(use_tc_tiling_on_sc=False),
    )
    return f(dstp, ones16, zeros16)


def _sc_scatter_body(y_ref, zeros_ref, src_ref, dst_ref, out_ref, sidx_v, didx_v,
                     rows_v, acc_sh, sem0, sem1):
    c = lax.axis_index("c")
    s = lax.axis_index("s")
    rbase = s * RPW
    @pl.when(c == 0)
    def _():
        pltpu.sync_copy(y_ref.at[pl.ds(rbase, RPW)], acc_sh.at[pl.ds(rbase, RPW)])

        @pl.when(s == NS - 1)
        def _():
            pltpu.sync_copy(y_ref.at[pl.ds(RPW * NS, RTAIL)],
                            acc_sh.at[pl.ds(RPW * NS, RTAIL)])

    @pl.when(c == 1)
    def _():
        pltpu.sync_copy(zeros_ref.at[pl.ds(rbase, RPW)], acc_sh.at[pl.ds(rbase, RPW)])

        @pl.when(s == NS - 1)
        def _():
            pltpu.sync_copy(zeros_ref.at[pl.ds(RPW * NS, RTAIL)],
                            acc_sh.at[pl.ds(RPW * NS, RTAIL)])

    nch_c = jnp.where(c == 0, NCH0, NCH1)
    base_row = jnp.where(c == 0, s * NCH0, NS * NCH0 + s * NCH1)
    pltpu.sync_copy(src_ref.at[pl.ds(base_row, NCHMAX)], sidx_v)
    pltpu.sync_copy(dst_ref.at[pl.ds(base_row, NCHMAX)], didx_v)
    plsc.subcore_barrier()

    sems = (sem0, sem1)
    pltpu.async_copy(y_ref.at[sidx_v.at[0]], rows_v.at[0], sem0)
    pltpu.async_copy(y_ref.at[sidx_v.at[1]], rows_v.at[1], sem1)

    def pair(i, carry):
        for b in range(2):
            k = i * 2 + b
            sem = sems[b]
            pltpu.make_async_copy(y_ref.at[sidx_v.at[0]], rows_v.at[b], sem).wait()
            pltpu.sync_copy(rows_v.at[b], acc_sh.at[didx_v.at[k]], add=True)

            @pl.when(k + 2 < nch_c)
            def _(k=k, b=b, sem=sem):
                pltpu.async_copy(y_ref.at[sidx_v.at[k + 2]], rows_v.at[b], sem)

        return carry

    lax.fori_loop(0, nch_c // 2, pair, 0)
    plsc.subcore_barrier()
    pltpu.sync_copy(acc_sh.at[pl.ds(rbase, RPW)], out_ref.at[pl.ds(c * N_NODES + rbase, RPW)])

    @pl.when(s == NS - 1)
    def _():
        pltpu.sync_copy(acc_sh.at[pl.ds(RPW * NS, RTAIL)],
                        out_ref.at[pl.ds(c * N_NODES + RPW * NS, RTAIL)])


def _sc_scatter(ybf, zbf, src_off, dstv):
    f = pl.kernel(
        _sc_scatter_body,
        out_type=jax.ShapeDtypeStruct((NC * N_NODES, D), jnp.bfloat16),
        mesh=plsc.VectorSubcoreMesh(core_axis_name="c", subcore_axis_name="s"),
        scratch_types=[
            pltpu.VMEM((NCHMAX, CH), jnp.int32),
            pltpu.VMEM((NCHMAX, CH), jnp.int32),
            pltpu.VMEM((2, CH, D), jnp.bfloat16),
            pltpu.VMEM_SHARED((AROWS, D), jnp.bfloat16),
            pltpu.SemaphoreType.DMA,
            pltpu.SemaphoreType.DMA,
        ],
        compiler_params=pltpu.CompilerParams(use_tc_tiling_on_sc=False,
                                             needs_layout_passes=False),
    )
    return f(ybf, zbf, src_off, dstv)


def _tc1_body(deg_ref, x_ref, w_ref, o_ref):
    dis = lax.rsqrt(deg_ref[...] + 1.0)
    xw = jnp.dot(x_ref[...], w_ref[...], preferred_element_type=jnp.float32)
    o_ref[...] = (xw * dis).astype(jnp.bfloat16)


def _tc1(degc, x, w):
    return pl.pallas_call(
        _tc1_body,
        grid=(GRID,),
        in_specs=[
            pl.BlockSpec((ROWBLK, 1), lambda i: (i, 0)),
            pl.BlockSpec((ROWBLK, D), lambda i: (i, 0)),
            pl.BlockSpec((D, D), lambda i: (0, 0)),
        ],
        out_specs=pl.BlockSpec((ROWBLK, D), lambda i: (i, 0)),
        out_shape=jax.ShapeDtypeStruct((N_NODES, D), jnp.bfloat16),
    )(degc, x, w)


def _tc2_body(deg_ref, a_ref, b_ref, w_ref, o_ref):
    dis = lax.rsqrt(deg_ref[...] + 1.0)
    acc = a_ref[0].astype(jnp.float32) + a_ref[1].astype(jnp.float32)
    h = jnp.maximum(acc * dis + b_ref[...], 0.0)
    xw = jnp.dot(h, w_ref[...], preferred_element_type=jnp.float32)
    o_ref[...] = (xw * dis).astype(jnp.bfloat16)


def _tc2(degc, accp, b, w):
    return pl.pallas_call(
        _tc2_body,
        grid=(GRID,),
        in_specs=[
            pl.BlockSpec((ROWBLK, 1), lambda i: (i, 0)),
            pl.BlockSpec((NC, ROWBLK, D), lambda i: (0, i, 0)),
            pl.BlockSpec((1, D), lambda i: (0, 0)),
            pl.BlockSpec((D, D), lambda i: (0, 0)),
        ],
        out_specs=pl.BlockSpec((ROWBLK, D), lambda i: (i, 0)),
        out_shape=jax.ShapeDtypeStruct((N_NODES, D), jnp.bfloat16),
    )(degc, accp, b, w)


def _tc3_body(deg_ref, a_ref, b_ref, o_ref):
    dis = lax.rsqrt(deg_ref[...] + 1.0)
    acc = a_ref[0].astype(jnp.float32) + a_ref[1].astype(jnp.float32)
    z = acc * dis + b_ref[...]
    m = jnp.max(z, axis=1, keepdims=True)
    e = jnp.exp(z - m)
    o_ref[...] = e / jnp.sum(e, axis=1, keepdims=True)


def _tc3(degc, accp, b):
    return pl.pallas_call(
        _tc3_body,
        grid=(GRID,),
        in_specs=[
            pl.BlockSpec((ROWBLK, 1), lambda i: (i, 0)),
            pl.BlockSpec((NC, ROWBLK, D), lambda i: (0, i, 0)),
            pl.BlockSpec((1, D), lambda i: (0, 0)),
        ],
        out_specs=pl.BlockSpec((ROWBLK, D), lambda i: (i, 0)),
        out_shape=jax.ShapeDtypeStruct((N_NODES, D), jnp.float32),
    )(degc, accp, b)


def kernel(feature, edge_index, W0, b0, W1, b1):
    src = edge_index[0]
    dst = edge_index[1]
    pad = EPAD - N_EDGES
    srcp = jnp.concatenate([src, jnp.zeros((pad,), src.dtype)])
    dstp = jnp.concatenate([dst, jnp.full((pad,), N_NODES, dst.dtype)])
    srcv = srcp.reshape(NC * NS * NCH, CH)
    dstv = dstp.reshape(NC * NS * NCH, CH)
    ones16 = jnp.ones((CH, 16), jnp.float32)
    zeros16 = jnp.zeros((DEGROWS, 16), jnp.float32)
    zbf = jnp.zeros((N_NODES, D), jnp.bfloat16)

    deg16 = _sc_degree(dstp, ones16, zeros16)
    degc = deg16[:, 0:1]

    y0 = _tc1(degc, feature, W0)
    acc0 = _sc_scatter(y0, zbf, srcv, dstv)
    y1 = _tc2(degc, acc0.reshape(NC, N_NODES, D), b0.reshape(1, D), W1)
    acc1 = _sc_scatter(y1, zbf, srcv, dstv)
    return _tc3(degc, acc1.reshape(NC, N_NODES, D), b1.reshape(1, D))
